# baseline (speedup 1.0000x reference)
import functools

import jax
import jax.numpy as jnp
from jax import lax
from jax.experimental import pallas as pl
from jax.experimental.pallas import tpu as pltpu
from jax.experimental.pallas import tpu_sc as plsc

D = 64
B = 16384
NC = 2
NS = 16
NW = NC * NS
BPW = B // NW
CH = 128
NCH = BPW // CH
LANES = 16

_mesh = plsc.VectorSubcoreMesh(core_axis_name="c", subcore_axis_name="s")


@functools.partial(
    pl.kernel,
    mesh=_mesh,
    out_type=jax.ShapeDtypeStruct((B, D), jnp.float32),
    scratch_types=[
        pltpu.VMEM((NCH, CH), jnp.int32),
        pltpu.VMEM((NCH, CH), jnp.int32),
        pltpu.VMEM((BPW, D), jnp.float32),
        pltpu.VMEM((BPW, D), jnp.float32),
        pltpu.SemaphoreType.DMA,
    ],
    compiler_params=pltpu.CompilerParams(use_tc_tiling_on_sc=False),
)
def _emb_kernel(sids_hbm, secs_hbm, stock_hbm, sector_hbm, out_hbm,
                sidx, cidx, bufs, bufc, sem):
    wid = lax.axis_index("s") * NC + lax.axis_index("c")
    base = wid * BPW

    pltpu.sync_copy(sids_hbm.at[wid], sidx)
    pltpu.sync_copy(secs_hbm.at[wid], cidx)

    copies = []
    for j in range(NCH):
        copies.append(pltpu.async_copy(
            stock_hbm.at[sidx.at[j]], bufs.at[pl.ds(j * CH, CH)], sem))
    for j in range(NCH):
        copies.append(pltpu.async_copy(
            sector_hbm.at[cidx.at[j]], bufc.at[pl.ds(j * CH, CH)], sem))
    for c in copies:
        c.wait()

    def body(r, carry):
        for c in range(D // LANES):
            sl = pl.ds(c * LANES, LANES)
            bufs[r, sl] = bufs[r, sl] + bufc[r, sl]
        return carry

    lax.fori_loop(0, BPW, body, 0)

    pltpu.sync_copy(bufs, out_hbm.at[pl.ds(base, BPW)])


def kernel(stock_ids, sector_ids, stock_table, sector_table):
    sids = stock_ids.astype(jnp.int32).reshape(NW, NCH, CH)
    secs = sector_ids.astype(jnp.int32).reshape(NW, NCH, CH)
    return _emb_kernel(sids, secs, stock_table, sector_table)
